# baseline (speedup 1.0000x reference)
import jax
import jax.numpy as jnp
from jax.experimental import pallas as pl


def _pdist(x1, x2):
    d2 = (jnp.sum(x1 * x1, axis=-1)[:, :, None]
          + jnp.sum(x2 * x2, axis=-1)[:, None, :]
          - 2.0 * jnp.einsum('bnd,bmd->bnm', x1, x2))
    return jnp.maximum(d2, 0.0)


def _auc(D, eps, iters):
    B, N, M = D.shape
    NEG = jnp.float32(-1e30)
    assignment = jnp.full((B, N), -1, dtype=jnp.int32)
    assignment_inv = jnp.full((B, M), -1, dtype=jnp.int32)
    price = jnp.zeros((B, M), dtype=jnp.float32)
    brow = jnp.arange(B)[:, None]
    irow = jnp.arange(N, dtype=jnp.int32)[None, :]

    def body(_, state):
        assignment, assignment_inv, price = state
        unass = assignment < 0
        value = -D - price[:, None, :]
        top2, top2_idx = jax.lax.top_k(value, 2)
        bid_item = top2_idx[:, :, 0].astype(jnp.int32)
        increment = top2[:, :, 0] - top2[:, :, 1] + eps
        scores = jnp.where(unass, increment, NEG)
        max_inc = jnp.full((B, M), NEG, dtype=jnp.float32).at[brow, bid_item].max(scores)
        cand = unass & (scores >= max_inc[brow, bid_item])
        win = jnp.full((B, M), N, dtype=jnp.int32).at[brow, bid_item].min(
            jnp.where(cand, irow, jnp.int32(N)))
        has_bid = win < N
        owned = jnp.where(assignment >= 0, assignment, 0)
        evicted = (assignment >= 0) & has_bid[brow, owned]
        is_winner = unass & (win[brow, bid_item] == irow)
        assignment = jnp.where(evicted, jnp.int32(-1),
                               jnp.where(is_winner, bid_item, assignment))
        assignment_inv = jnp.where(has_bid, win, assignment_inv)
        price = price + jnp.where(has_bid, max_inc, 0.0)
        return assignment, assignment_inv, price

    assignment, assignment_inv, price = jax.lax.fori_loop(
        0, iters, body, (assignment, assignment_inv, price))
    return assignment


def _passthrough(d_ref, a_ref, do_ref, ao_ref):
    do_ref[...] = d_ref[...]
    ao_ref[...] = a_ref[...]


def kernel(input1, input2, ass, ass_inv, price, eps, iters):
    x1 = input1.astype(jnp.float32)
    x2 = input2.astype(jnp.float32)
    D = _pdist(x1, x2)
    assignment = _auc(D, jnp.float32(eps), iters)
    a = jnp.maximum(assignment, 0)
    gathered = jnp.take_along_axis(x2, a[:, :, None].astype(jnp.int32), axis=1)
    dist = jnp.sum((x1 - gathered) ** 2, axis=-1)
    dist = jnp.where(assignment >= 0, dist, 0.0)
    dist, assignment = pl.pallas_call(
        _passthrough,
        out_shape=(jax.ShapeDtypeStruct(dist.shape, dist.dtype),
                   jax.ShapeDtypeStruct(assignment.shape, assignment.dtype)),
    )(dist, assignment)
    return dist, assignment


# speedup vs baseline: 45.4358x; 45.4358x over previous
import functools

import jax
import jax.numpy as jnp
from jax import lax
from jax.experimental import pallas as pl
from jax.experimental.pallas import tpu as pltpu
from jax.experimental.pallas import tpu_sc as plsc

B, N = 8, 1024
NCH = N // 16
NEG = -1e30
FMIN = -3.0e38


def _sc_auction(negd, x1f, x2f, epsv, itersv):
    mesh = plsc.VectorSubcoreMesh(core_axis_name="c", subcore_axis_name="s")

    @functools.partial(
        pl.kernel,
        mesh=mesh,
        compiler_params=pltpu.CompilerParams(needs_layout_passes=False),
        out_type=(jax.ShapeDtypeStruct((B, N), jnp.float32),
                  jax.ShapeDtypeStruct((B, N), jnp.int32)),
        scratch_types=[
            pltpu.VMEM((16, N), jnp.float32),
            pltpu.VMEM((N,), jnp.float32),
            pltpu.VMEM((N,), jnp.int32),
            pltpu.VMEM((N,), jnp.int32),
            pltpu.VMEM((N,), jnp.float32),
            pltpu.VMEM((N,), jnp.int32),
            pltpu.VMEM((N + 16,), jnp.int32),
            pltpu.VMEM((N + 16,), jnp.int32),
            pltpu.VMEM((N,), jnp.int32),
            pltpu.VMEM((3 * N,), jnp.float32),
            pltpu.VMEM((3 * N,), jnp.float32),
            pltpu.VMEM((16,), jnp.float32),
            pltpu.VMEM((16,), jnp.int32),
            pltpu.VMEM((N,), jnp.float32),
            pltpu.SemaphoreType.DMA,
        ],
    )
    def k(negd_h, x1f_h, x2f_h, eps_h, iters_h, dist_o, ass_o,
          rowbuf, price, assign, ainv, maxinc, winv, lista, listb, bids,
          x1l, x2l, epsl, itersl, distv, sem):
        cid = lax.axis_index("c")
        sid = lax.axis_index("s")
        b = sid * 2 + cid

        @pl.when(b < B)
        def _():
            lane = lax.broadcasted_iota(jnp.int32, (16,), 0)
            lane0 = lane == 0
            pltpu.sync_copy(x1f_h.at[b], x1l)
            pltpu.sync_copy(x2f_h.at[b], x2l)
            pltpu.sync_copy(eps_h, epsl)
            pltpu.sync_copy(iters_h, itersl)
            eps_s = jnp.max(epsl[...])
            iters_s = jnp.max(itersl[...])
            zf = jnp.zeros((16,), jnp.float32)
            neg1 = jnp.full((16,), -1, jnp.int32)
            for ci in range(NCH):
                sl = pl.ds(ci * 16, 16)
                price[sl] = zf
                assign[sl] = neg1
                ainv[sl] = neg1
                lista[sl] = lane + (16 * ci)
            zi = jnp.zeros((16,), jnp.int32)
            lista[pl.ds(N, 16)] = zi
            for ci in range(NCH + 1):
                listb[pl.ds(ci * 16, 16)] = zi
            boff = b * N

            def iter_body(_, count):
                for ci in range(NCH):
                    sl = pl.ds(ci * 16, 16)
                    maxinc[sl] = jnp.full((16,), NEG, jnp.float32)
                    winv[sl] = jnp.full((16,), N, jnp.int32)
                ngroups = (count + 15) // 16

                def group_body(g, carry):
                    base = g * 16
                    n16 = plsc.load_gather(lista, [lane + base])
                    pltpu.async_copy(negd_h.at[n16 + boff], rowbuf, sem).wait()
                    gsize = jnp.minimum(count - base, 16)

                    def bidder_body(j, carry2):
                        kpos = base + j
                        n_v = plsc.load_gather(lista, [jnp.full((16,), kpos, jnp.int32)])
                        jv = jnp.full((16,), j, jnp.int32)
                        m1 = jnp.full((16,), FMIN, jnp.float32)
                        m2 = jnp.full((16,), FMIN, jnp.float32)
                        i1 = jnp.zeros((16,), jnp.int32)
                        for ci2 in range(NCH):
                            idxc = lane + (16 * ci2)
                            nd = plsc.load_gather(rowbuf, [jv, idxc])
                            pr = price[pl.ds(16 * ci2, 16)]
                            v = nd - pr
                            gt = v > m1
                            m2 = jnp.where(gt, m1, jnp.maximum(m2, v))
                            i1 = jnp.where(gt, idxc, i1)
                            m1 = jnp.where(gt, v, m1)
                        g1 = jnp.max(m1)
                        lm = m1 == g1
                        bid_s = jnp.min(jnp.where(lm, i1, jnp.int32(4 * N)))
                        chosen = lm & (i1 == bid_s)
                        second = jnp.max(jnp.where(chosen, m2, m1))
                        inc_s = (g1 - second) + eps_s
                        inc_v = jnp.full((16,), inc_s, jnp.float32)
                        bid_v = jnp.full((16,), bid_s, jnp.int32)
                        cur = plsc.load_gather(maxinc, [bid_v])
                        wcur = plsc.load_gather(winv, [bid_v])
                        take = inc_v > cur
                        tie = inc_v == cur
                        nwin = jnp.where(take, n_v,
                                         jnp.where(tie, jnp.minimum(wcur, n_v), wcur))
                        plsc.store_scatter(maxinc, [bid_v],
                                           jnp.maximum(inc_v, cur), mask=lane0)
                        plsc.store_scatter(winv, [bid_v], nwin, mask=lane0)
                        plsc.store_scatter(bids, [jnp.full((16,), kpos, jnp.int32)],
                                           bid_v, mask=lane0)
                        return carry2

                    lax.fori_loop(0, gsize, bidder_body, 0)
                    return carry

                lax.fori_loop(0, ngroups, group_body, 0)

                def apply_body(kc, cnt2):
                    kidx = lane + kc * 16
                    valid = kidx < count
                    n16 = plsc.load_gather(lista, [kidx])
                    b16 = plsc.load_gather(bids, [kidx])
                    w16 = plsc.load_gather(winv, [b16])
                    iswin = valid & (w16 == n16)
                    old = plsc.load_gather(ainv, [b16])
                    evict = iswin & (old >= 0)
                    plsc.store_scatter(assign, [jnp.maximum(old, 0)], neg1, mask=evict)
                    plsc.store_scatter(assign, [n16], b16, mask=iswin)
                    plsc.store_scatter(ainv, [b16], n16, mask=iswin)
                    pcur = plsc.load_gather(price, [b16])
                    minc = plsc.load_gather(maxinc, [b16])
                    plsc.store_scatter(price, [b16], pcur + minc, mask=iswin)
                    lose = valid & jnp.logical_not(iswin)
                    lc = lax.cumsum(lose.astype(jnp.int32))
                    plsc.store_scatter(listb, [cnt2 + lc - 1], n16, mask=lose)
                    cnt2 = cnt2 + jnp.max(lc)
                    ec = lax.cumsum(evict.astype(jnp.int32))
                    plsc.store_scatter(listb, [cnt2 + ec - 1],
                                       jnp.maximum(old, 0), mask=evict)
                    cnt2 = cnt2 + jnp.max(ec)
                    return cnt2

                count2 = lax.fori_loop(0, ngroups, apply_body, jnp.int32(0))
                for ci in range(NCH + 1):
                    sl = pl.ds(ci * 16, 16)
                    lista[sl] = listb[sl]
                return count2

            lax.fori_loop(0, iters_s, iter_body, jnp.int32(N))

            for ci in range(NCH):
                sl = pl.ds(ci * 16, 16)
                a = assign[sl]
                idx = jnp.maximum(a, 0)
                gx = plsc.load_gather(x2l, [idx])
                gy = plsc.load_gather(x2l, [idx + N])
                gz = plsc.load_gather(x2l, [idx + 2 * N])
                dx = x1l[sl] - gx
                dy = x1l[pl.ds(N + ci * 16, 16)] - gy
                dz = x1l[pl.ds(2 * N + ci * 16, 16)] - gz
                dd = (dx * dx + dy * dy) + dz * dz
                distv[sl] = jnp.where(a < 0, jnp.float32(0.0), dd)
            pltpu.sync_copy(distv, dist_o.at[b])
            pltpu.sync_copy(assign, ass_o.at[b])

    return k(negd, x1f, x2f, epsv, itersv)


def kernel(input1, input2, ass, ass_inv, price, eps, iters):
    x1 = input1.astype(jnp.float32)
    x2 = input2.astype(jnp.float32)
    d2 = (jnp.sum(x1 * x1, axis=-1)[:, :, None]
          + jnp.sum(x2 * x2, axis=-1)[:, None, :]
          - 2.0 * jnp.einsum('bnd,bmd->bnm', x1, x2))
    D = jnp.maximum(d2, 0.0)
    negd = (-D).reshape(B * N, N)
    x1f = jnp.transpose(x1, (0, 2, 1)).reshape(B, 3 * N)
    x2f = jnp.transpose(x2, (0, 2, 1)).reshape(B, 3 * N)
    epsv = jnp.full((16,), eps, jnp.float32)
    itersv = jnp.full((16,), iters, jnp.int32)
    dist, assignment = _sc_auction(negd, x1f, x2f, epsv, itersv)
    return dist, assignment


# speedup vs baseline: 62.6851x; 1.3796x over previous
import functools

import jax
import jax.numpy as jnp
from jax import lax
from jax.experimental import pallas as pl
from jax.experimental.pallas import tpu as pltpu
from jax.experimental.pallas import tpu_sc as plsc

B, N = 8, 1024
NCH = N // 16
NEG = -1e30
FMIN = -3.0e38


def _sc_auction(negd, x1f, x2f, epsv, itersv):
    mesh = plsc.VectorSubcoreMesh(core_axis_name="c", subcore_axis_name="s")

    @functools.partial(
        pl.kernel,
        mesh=mesh,
        compiler_params=pltpu.CompilerParams(needs_layout_passes=False),
        out_type=(jax.ShapeDtypeStruct((B, N), jnp.float32),
                  jax.ShapeDtypeStruct((B, N), jnp.int32)),
        scratch_types=[
            pltpu.VMEM((2, 16, N), jnp.float32),
            pltpu.VMEM((N,), jnp.float32),
            pltpu.VMEM((N,), jnp.int32),
            pltpu.VMEM((N,), jnp.int32),
            pltpu.VMEM((N,), jnp.float32),
            pltpu.VMEM((N,), jnp.int32),
            pltpu.VMEM((N + 16,), jnp.int32),
            pltpu.VMEM((N + 16,), jnp.int32),
            pltpu.VMEM((N,), jnp.int32),
            pltpu.VMEM((3 * N,), jnp.float32),
            pltpu.VMEM((3 * N,), jnp.float32),
            pltpu.VMEM((16,), jnp.float32),
            pltpu.VMEM((16,), jnp.int32),
            pltpu.VMEM((N,), jnp.float32),
            pltpu.SemaphoreType.DMA,
        ],
    )
    def k(negd_h, x1f_h, x2f_h, eps_h, iters_h, dist_o, ass_o,
          rowbuf, price, assign, ainv, maxinc, winv, lista, listb, bids,
          x1l, x2l, epsl, itersl, distv, sem):
        cid = lax.axis_index("c")
        sid = lax.axis_index("s")
        b = sid * 2 + cid

        @pl.when(b < B)
        def _():
            lane = lax.broadcasted_iota(jnp.int32, (16,), 0)
            lane0 = lane == 0
            pltpu.sync_copy(x1f_h.at[b], x1l)
            pltpu.sync_copy(x2f_h.at[b], x2l)
            pltpu.sync_copy(eps_h, epsl)
            pltpu.sync_copy(iters_h, itersl)
            eps_s = jnp.max(epsl[...])
            iters_s = jnp.max(itersl[...])
            zf = jnp.zeros((16,), jnp.float32)
            neg1 = jnp.full((16,), -1, jnp.int32)
            for ci in range(NCH):
                sl = pl.ds(ci * 16, 16)
                price[sl] = zf
                assign[sl] = neg1
                ainv[sl] = neg1
                lista[sl] = lane + (16 * ci)
            zi = jnp.zeros((16,), jnp.int32)
            lista[pl.ds(N, 16)] = zi
            for ci in range(NCH + 1):
                listb[pl.ds(ci * 16, 16)] = zi
            boff = b * N

            def iter_body(_, count):
                for ci in range(NCH):
                    sl = pl.ds(ci * 16, 16)
                    maxinc[sl] = jnp.full((16,), NEG, jnp.float32)
                    winv[sl] = jnp.full((16,), N, jnp.int32)
                ngroups = (count + 15) // 16

                def issue(g):
                    n16 = plsc.load_gather(lista, [lane + g * 16])
                    pltpu.async_copy(negd_h.at[n16 + boff], rowbuf.at[g % 2], sem)

                @pl.when(ngroups > 0)
                def _():
                    issue(jnp.int32(0))

                def group_body(g, carry):
                    base = g * 16
                    slot = g % 2
                    pltpu.make_async_copy(negd_h.at[lane], rowbuf.at[slot], sem).wait()

                    @pl.when(g + 1 < ngroups)
                    def _():
                        issue(g + 1)

                    slotv = jnp.full((16,), slot, jnp.int32)
                    gsize = jnp.minimum(count - base, 16)

                    def bidder_body(j, carry2):
                        kpos = base + j
                        n_v = plsc.load_gather(lista, [jnp.full((16,), kpos, jnp.int32)])
                        jv = jnp.full((16,), j, jnp.int32)
                        m1 = jnp.full((16,), FMIN, jnp.float32)
                        m2 = jnp.full((16,), FMIN, jnp.float32)
                        i1 = jnp.zeros((16,), jnp.int32)
                        for ci2 in range(NCH):
                            idxc = lane + (16 * ci2)
                            nd = plsc.load_gather(rowbuf, [slotv, jv, idxc])
                            pr = price[pl.ds(16 * ci2, 16)]
                            v = nd - pr
                            gt = v > m1
                            m2 = jnp.where(gt, m1, jnp.maximum(m2, v))
                            i1 = jnp.where(gt, idxc, i1)
                            m1 = jnp.where(gt, v, m1)
                        g1 = jnp.max(m1)
                        lm = m1 == g1
                        bid_s = jnp.min(jnp.where(lm, i1, jnp.int32(4 * N)))
                        chosen = lm & (i1 == bid_s)
                        second = jnp.max(jnp.where(chosen, m2, m1))
                        inc_s = (g1 - second) + eps_s
                        inc_v = jnp.full((16,), inc_s, jnp.float32)
                        bid_v = jnp.full((16,), bid_s, jnp.int32)
                        cur = plsc.load_gather(maxinc, [bid_v])
                        wcur = plsc.load_gather(winv, [bid_v])
                        take = inc_v > cur
                        tie = inc_v == cur
                        nwin = jnp.where(take, n_v,
                                         jnp.where(tie, jnp.minimum(wcur, n_v), wcur))
                        plsc.store_scatter(maxinc, [bid_v],
                                           jnp.maximum(inc_v, cur), mask=lane0)
                        plsc.store_scatter(winv, [bid_v], nwin, mask=lane0)
                        plsc.store_scatter(bids, [jnp.full((16,), kpos, jnp.int32)],
                                           bid_v, mask=lane0)
                        return carry2

                    lax.fori_loop(0, gsize, bidder_body, 0)
                    return carry

                lax.fori_loop(0, ngroups, group_body, 0)

                def apply_body(kc, cnt2):
                    kidx = lane + kc * 16
                    valid = kidx < count
                    n16 = plsc.load_gather(lista, [kidx])
                    b16 = plsc.load_gather(bids, [kidx])
                    w16 = plsc.load_gather(winv, [b16])
                    iswin = valid & (w16 == n16)
                    old = plsc.load_gather(ainv, [b16])
                    evict = iswin & (old >= 0)
                    plsc.store_scatter(assign, [jnp.maximum(old, 0)], neg1, mask=evict)
                    plsc.store_scatter(assign, [n16], b16, mask=iswin)
                    plsc.store_scatter(ainv, [b16], n16, mask=iswin)
                    pcur = plsc.load_gather(price, [b16])
                    minc = plsc.load_gather(maxinc, [b16])
                    plsc.store_scatter(price, [b16], pcur + minc, mask=iswin)
                    lose = valid & jnp.logical_not(iswin)
                    lc = lax.cumsum(lose.astype(jnp.int32))
                    plsc.store_scatter(listb, [cnt2 + lc - 1], n16, mask=lose)
                    cnt2 = cnt2 + jnp.max(lc)
                    ec = lax.cumsum(evict.astype(jnp.int32))
                    plsc.store_scatter(listb, [cnt2 + ec - 1],
                                       jnp.maximum(old, 0), mask=evict)
                    cnt2 = cnt2 + jnp.max(ec)
                    return cnt2

                count2 = lax.fori_loop(0, ngroups, apply_body, jnp.int32(0))
                for ci in range(NCH + 1):
                    sl = pl.ds(ci * 16, 16)
                    lista[sl] = listb[sl]
                return count2

            lax.fori_loop(0, iters_s, iter_body, jnp.int32(N))

            for ci in range(NCH):
                sl = pl.ds(ci * 16, 16)
                a = assign[sl]
                idx = jnp.maximum(a, 0)
                gx = plsc.load_gather(x2l, [idx])
                gy = plsc.load_gather(x2l, [idx + N])
                gz = plsc.load_gather(x2l, [idx + 2 * N])
                dx = x1l[sl] - gx
                dy = x1l[pl.ds(N + ci * 16, 16)] - gy
                dz = x1l[pl.ds(2 * N + ci * 16, 16)] - gz
                dd = (dx * dx + dy * dy) + dz * dz
                distv[sl] = jnp.where(a < 0, jnp.float32(0.0), dd)
            pltpu.sync_copy(distv, dist_o.at[b])
            pltpu.sync_copy(assign, ass_o.at[b])

    return k(negd, x1f, x2f, epsv, itersv)


def kernel(input1, input2, ass, ass_inv, price, eps, iters):
    x1 = input1.astype(jnp.float32)
    x2 = input2.astype(jnp.float32)
    d2 = (jnp.sum(x1 * x1, axis=-1)[:, :, None]
          + jnp.sum(x2 * x2, axis=-1)[:, None, :]
          - 2.0 * jnp.einsum('bnd,bmd->bnm', x1, x2))
    D = jnp.maximum(d2, 0.0)
    negd = (-D).reshape(B * N, N)
    x1f = jnp.transpose(x1, (0, 2, 1)).reshape(B, 3 * N)
    x2f = jnp.transpose(x2, (0, 2, 1)).reshape(B, 3 * N)
    epsv = jnp.full((16,), eps, jnp.float32)
    itersv = jnp.full((16,), iters, jnp.int32)
    dist, assignment = _sc_auction(negd, x1f, x2f, epsv, itersv)
    return dist, assignment


# speedup vs baseline: 71.5302x; 1.1411x over previous
import functools

import jax
import jax.numpy as jnp
from jax import lax
from jax.experimental import pallas as pl
from jax.experimental.pallas import tpu as pltpu
from jax.experimental.pallas import tpu_sc as plsc

B, N = 8, 1024
NCH = N // 16
NEG = -1e30
FMIN = -3.0e38


def _sc_auction(negd, x1f, x2f, epsv, itersv):
    mesh = plsc.VectorSubcoreMesh(core_axis_name="c", subcore_axis_name="s")

    @functools.partial(
        pl.kernel,
        mesh=mesh,
        compiler_params=pltpu.CompilerParams(needs_layout_passes=False),
        out_type=(jax.ShapeDtypeStruct((B, N), jnp.float32),
                  jax.ShapeDtypeStruct((B, N), jnp.int32),
                  jax.ShapeDtypeStruct((4 * B, N), jnp.float32),
                  jax.ShapeDtypeStruct((4 * B, N), jnp.int32)),
        scratch_types=[
            pltpu.VMEM((2, 16, N), jnp.float32),
            pltpu.VMEM((N,), jnp.float32),
            pltpu.VMEM((N,), jnp.int32),
            pltpu.VMEM((N,), jnp.int32),
            pltpu.VMEM((N,), jnp.float32),
            pltpu.VMEM((N,), jnp.int32),
            pltpu.VMEM((N + 16,), jnp.int32),
            pltpu.VMEM((4, N), jnp.float32),
            pltpu.VMEM((4, N), jnp.int32),
            pltpu.VMEM((3 * N,), jnp.float32),
            pltpu.VMEM((3 * N,), jnp.float32),
            pltpu.VMEM((16,), jnp.float32),
            pltpu.VMEM((16,), jnp.int32),
            pltpu.VMEM((N,), jnp.float32),
            pltpu.SemaphoreType.DMA,
        ],
    )
    def k(negd_h, x1f_h, x2f_h, eps_h, iters_h, dist_o, ass_o, minc_h, win_h,
          rowbuf, price, assign, ainv, maxinc, winv, lista,
          minc_all, win_all, x1l, x2l, epsl, itersl, distv, sem):
        cid = lax.axis_index("c")
        sid = lax.axis_index("s")
        b = cid * 4 + sid // 4
        r = sid % 4

        lane = lax.broadcasted_iota(jnp.int32, (16,), 0)
        pltpu.sync_copy(x1f_h.at[b], x1l)
        pltpu.sync_copy(x2f_h.at[b], x2l)
        pltpu.sync_copy(eps_h, epsl)
        pltpu.sync_copy(iters_h, itersl)
        eps_s = jnp.max(epsl[...])
        iters_s = jnp.max(itersl[...])
        zf = jnp.zeros((16,), jnp.float32)
        neg1 = jnp.full((16,), -1, jnp.int32)
        zi = jnp.zeros((16,), jnp.int32)
        boff = b * N

        for ci in range(NCH):
            sl = pl.ds(ci * 16, 16)
            price[sl] = zf
            assign[sl] = neg1
            ainv[sl] = neg1
            lista[sl] = lane + (16 * ci)
        lista[pl.ds(N, 16)] = zi

        def iter_body(_, count):
            for ci in range(NCH):
                sl = pl.ds(ci * 16, 16)
                maxinc[sl] = jnp.full((16,), NEG, jnp.float32)
                winv[sl] = jnp.full((16,), N, jnp.int32)

            qsz = (count + 3) // 4
            klo = r * qsz
            kcnt = jnp.maximum(jnp.minimum(qsz, count - klo), 0)
            ngroups = (kcnt + 15) // 16

            def issue(g):
                n16 = plsc.load_gather(lista, [lane + (klo + g * 16)])
                pltpu.async_copy(negd_h.at[n16 + boff], rowbuf.at[g % 2], sem)

            @pl.when(ngroups > 0)
            def _():
                issue(jnp.int32(0))

            def group_body(g, carry2):
                base = klo + g * 16
                slot = g % 2
                pltpu.make_async_copy(negd_h.at[lane], rowbuf.at[slot], sem).wait()

                @pl.when(g + 1 < ngroups)
                def _():
                    issue(g + 1)

                slotv = jnp.full((16,), slot, jnp.int32)
                gsize = jnp.minimum(kcnt - g * 16, 16)

                def bidder_body(j, carry3):
                    n_v = plsc.load_gather(
                        lista, [jnp.full((16,), base + j, jnp.int32)])
                    jv = jnp.full((16,), j, jnp.int32)
                    m1 = jnp.full((16,), FMIN, jnp.float32)
                    m2 = jnp.full((16,), FMIN, jnp.float32)
                    i1 = jnp.zeros((16,), jnp.int32)
                    for ci2 in range(NCH):
                        idxc = lane + (16 * ci2)
                        nd = plsc.load_gather(rowbuf, [slotv, jv, idxc])
                        pr = price[pl.ds(16 * ci2, 16)]
                        v = nd - pr
                        gt = v > m1
                        m2 = jnp.where(gt, m1, jnp.maximum(m2, v))
                        i1 = jnp.where(gt, idxc, i1)
                        m1 = jnp.where(gt, v, m1)
                    g1 = jnp.max(m1)
                    lm = m1 == g1
                    bid_s = jnp.min(jnp.where(lm, i1, jnp.int32(4 * N)))
                    chosen = lm & (i1 == bid_s)
                    second = jnp.max(jnp.where(chosen, m2, m1))
                    inc_s = (g1 - second) + eps_s
                    inc_v = jnp.full((16,), inc_s, jnp.float32)
                    bid_v = jnp.full((16,), bid_s, jnp.int32)
                    cur = plsc.load_gather(maxinc, [bid_v])
                    wcur = plsc.load_gather(winv, [bid_v])
                    take = inc_v > cur
                    tie = inc_v == cur
                    nwin = jnp.where(take, n_v,
                                     jnp.where(tie, jnp.minimum(wcur, n_v), wcur))
                    lane0 = lane == 0
                    plsc.store_scatter(maxinc, [bid_v],
                                       jnp.maximum(inc_v, cur), mask=lane0)
                    plsc.store_scatter(winv, [bid_v], nwin, mask=lane0)
                    return carry3

                lax.fori_loop(0, gsize, bidder_body, 0)
                return carry2

            lax.fori_loop(0, ngroups, group_body, 0)

            pltpu.sync_copy(maxinc, minc_h.at[b * 4 + r])
            pltpu.sync_copy(winv, win_h.at[b * 4 + r])
            plsc.subcore_barrier()
            for ra in range(4):
                pltpu.sync_copy(minc_h.at[b * 4 + ra], minc_all.at[ra])
                pltpu.sync_copy(win_h.at[b * 4 + ra], win_all.at[ra])
            cnt2 = jnp.int32(0)
            for ci in range(NCH):
                sl = pl.ds(ci * 16, 16)
                mm = minc_all[0, sl]
                ww = win_all[0, sl]
                for ra in range(1, 4):
                    mo = minc_all[ra, sl]
                    wo = win_all[ra, sl]
                    gt = mm > mo
                    eq = mm == mo
                    ww = jnp.where(gt, ww,
                                   jnp.where(eq, jnp.minimum(ww, wo), wo))
                    mm = jnp.maximum(mm, mo)
                h = ww < N
                old = ainv[sl]
                ev = h & (old >= 0)
                plsc.store_scatter(assign, [jnp.maximum(old, 0)], neg1,
                                   mask=ev)
                idxc = lane + (16 * ci)
                plsc.store_scatter(assign, [jnp.where(h, ww, 0)], idxc,
                                   mask=h)
                ainv[sl] = jnp.where(h, ww, old)
                price[sl] = price[sl] + jnp.where(h, mm, jnp.float32(0.0))
            for ci in range(NCH):
                sl = pl.ds(ci * 16, 16)
                u = assign[sl] < 0
                lc = lax.cumsum(u.astype(jnp.int32))
                plsc.store_scatter(lista, [cnt2 + lc - 1],
                                   lane + (16 * ci), mask=u)
                cnt2 = cnt2 + jnp.max(lc)
            plsc.subcore_barrier()
            return cnt2

        lax.fori_loop(0, iters_s, iter_body, jnp.int32(N))

        @pl.when(r == 0)
        def _():
            for ci in range(NCH):
                sl = pl.ds(ci * 16, 16)
                a = assign[sl]
                idx = jnp.maximum(a, 0)
                gx = plsc.load_gather(x2l, [idx])
                gy = plsc.load_gather(x2l, [idx + N])
                gz = plsc.load_gather(x2l, [idx + 2 * N])
                dx = x1l[sl] - gx
                dy = x1l[pl.ds(N + ci * 16, 16)] - gy
                dz = x1l[pl.ds(2 * N + ci * 16, 16)] - gz
                dd = (dx * dx + dy * dy) + dz * dz
                distv[sl] = jnp.where(a < 0, jnp.float32(0.0), dd)
            pltpu.sync_copy(distv, dist_o.at[b])
            pltpu.sync_copy(assign, ass_o.at[b])

    return k(negd, x1f, x2f, epsv, itersv)


def kernel(input1, input2, ass, ass_inv, price, eps, iters):
    x1 = input1.astype(jnp.float32)
    x2 = input2.astype(jnp.float32)
    d2 = (jnp.sum(x1 * x1, axis=-1)[:, :, None]
          + jnp.sum(x2 * x2, axis=-1)[:, None, :]
          - 2.0 * jnp.einsum('bnd,bmd->bnm', x1, x2))
    D = jnp.maximum(d2, 0.0)
    negd = (-D).reshape(B * N, N)
    x1f = jnp.transpose(x1, (0, 2, 1)).reshape(B, 3 * N)
    x2f = jnp.transpose(x2, (0, 2, 1)).reshape(B, 3 * N)
    epsv = jnp.full((16,), eps, jnp.float32)
    itersv = jnp.full((16,), iters, jnp.int32)
    dist, assignment, _, _ = _sc_auction(negd, x1f, x2f, epsv, itersv)
    return dist, assignment


# speedup vs baseline: 96.2857x; 1.3461x over previous
import functools

import jax
import jax.numpy as jnp
from jax import lax
from jax.experimental import pallas as pl
from jax.experimental.pallas import tpu as pltpu
from jax.experimental.pallas import tpu_sc as plsc

B, N = 8, 1024
NCH = N // 16
NEG = -1e30
FMIN = -3.0e38


def _sc_auction(negd, x1f, x2f, epsv, itersv):
    mesh = plsc.VectorSubcoreMesh(core_axis_name="c", subcore_axis_name="s")

    @functools.partial(
        pl.kernel,
        mesh=mesh,
        compiler_params=pltpu.CompilerParams(needs_layout_passes=False),
        out_type=(jax.ShapeDtypeStruct((B, N), jnp.float32),
                  jax.ShapeDtypeStruct((B, N), jnp.int32),
                  jax.ShapeDtypeStruct((8 * B, N), jnp.float32),
                  jax.ShapeDtypeStruct((8 * B, N), jnp.int32)),
        scratch_types=[
            pltpu.VMEM((2, 16, N), jnp.float32),
            pltpu.VMEM((N,), jnp.float32),
            pltpu.VMEM((N,), jnp.int32),
            pltpu.VMEM((N,), jnp.int32),
            pltpu.VMEM((N,), jnp.float32),
            pltpu.VMEM((N,), jnp.int32),
            pltpu.VMEM((N + 16,), jnp.int32),
            pltpu.VMEM((4, N), jnp.float32),
            pltpu.VMEM((4, N), jnp.int32),
            pltpu.VMEM((3 * N,), jnp.float32),
            pltpu.VMEM((3 * N,), jnp.float32),
            pltpu.VMEM((16,), jnp.float32),
            pltpu.VMEM((16,), jnp.int32),
            pltpu.VMEM((N,), jnp.float32),
            pltpu.SemaphoreType.DMA,
        ],
    )
    def k(negd_h, x1f_h, x2f_h, eps_h, iters_h, dist_o, ass_o, minc_h, win_h,
          rowbuf, price, assign, ainv, maxinc, winv, lista,
          minc_all, win_all, x1l, x2l, epsl, itersl, distv, sem):
        cid = lax.axis_index("c")
        sid = lax.axis_index("s")
        b = cid * 4 + sid // 4
        r = sid % 4

        lane = lax.broadcasted_iota(jnp.int32, (16,), 0)
        pltpu.sync_copy(x1f_h.at[b], x1l)
        pltpu.sync_copy(x2f_h.at[b], x2l)
        pltpu.sync_copy(eps_h, epsl)
        pltpu.sync_copy(iters_h, itersl)
        eps_s = jnp.max(epsl[...])
        iters_s = jnp.max(itersl[...])
        zf = jnp.zeros((16,), jnp.float32)
        neg1 = jnp.full((16,), -1, jnp.int32)
        zi = jnp.zeros((16,), jnp.int32)
        boff = b * N

        for ci in range(NCH):
            sl = pl.ds(ci * 16, 16)
            price[sl] = zf
            assign[sl] = neg1
            ainv[sl] = neg1
            lista[sl] = lane + (16 * ci)
        lista[pl.ds(N, 16)] = zi

        def iter_body(it, count):
            pslot = (it % 2) * (4 * B)
            for ci in range(NCH):
                sl = pl.ds(ci * 16, 16)
                maxinc[sl] = jnp.full((16,), NEG, jnp.float32)
                winv[sl] = jnp.full((16,), N, jnp.int32)

            qsz = (count + 3) // 4
            klo = r * qsz
            kcnt = jnp.maximum(jnp.minimum(qsz, count - klo), 0)
            ngroups = (kcnt + 15) // 16

            def issue(g):
                n16 = plsc.load_gather(lista, [lane + (klo + g * 16)])
                pltpu.async_copy(negd_h.at[n16 + boff], rowbuf.at[g % 2], sem)

            @pl.when(ngroups > 0)
            def _():
                issue(jnp.int32(0))

            def group_body(g, carry2):
                base = klo + g * 16
                slot = g % 2
                pltpu.make_async_copy(negd_h.at[lane], rowbuf.at[slot], sem).wait()

                @pl.when(g + 1 < ngroups)
                def _():
                    issue(g + 1)

                slotv = jnp.full((16,), slot, jnp.int32)
                gsize = jnp.minimum(kcnt - g * 16, 16)

                def bidder_body(j, carry3):
                    n_v = plsc.load_gather(
                        lista, [jnp.full((16,), base + j, jnp.int32)])
                    jv = jnp.full((16,), j, jnp.int32)
                    m1 = jnp.full((16,), FMIN, jnp.float32)
                    m2 = jnp.full((16,), FMIN, jnp.float32)
                    i1 = jnp.zeros((16,), jnp.int32)
                    for ci2 in range(NCH):
                        idxc = lane + (16 * ci2)
                        nd = plsc.load_gather(rowbuf, [slotv, jv, idxc])
                        pr = price[pl.ds(16 * ci2, 16)]
                        v = nd - pr
                        gt = v > m1
                        m2 = jnp.where(gt, m1, jnp.maximum(m2, v))
                        i1 = jnp.where(gt, idxc, i1)
                        m1 = jnp.where(gt, v, m1)
                    g1 = jnp.max(m1)
                    lm = m1 == g1
                    bid_s = jnp.min(jnp.where(lm, i1, jnp.int32(4 * N)))
                    chosen = lm & (i1 == bid_s)
                    second = jnp.max(jnp.where(chosen, m2, m1))
                    inc_s = (g1 - second) + eps_s
                    inc_v = jnp.full((16,), inc_s, jnp.float32)
                    bid_v = jnp.full((16,), bid_s, jnp.int32)
                    cur = plsc.load_gather(maxinc, [bid_v])
                    wcur = plsc.load_gather(winv, [bid_v])
                    take = inc_v > cur
                    tie = inc_v == cur
                    nwin = jnp.where(take, n_v,
                                     jnp.where(tie, jnp.minimum(wcur, n_v), wcur))
                    lane0 = lane == 0
                    plsc.store_scatter(maxinc, [bid_v],
                                       jnp.maximum(inc_v, cur), mask=lane0)
                    plsc.store_scatter(winv, [bid_v], nwin, mask=lane0)
                    return carry3

                lax.fori_loop(0, gsize, bidder_body, 0)
                return carry2

            lax.fori_loop(0, ngroups, group_body, 0)

            pltpu.async_copy(maxinc, minc_h.at[pslot + b * 4 + r], sem)
            pltpu.make_async_copy(maxinc, minc_h.at[pslot + b * 4 + r], sem).wait()
            pltpu.async_copy(winv, win_h.at[pslot + b * 4 + r], sem)
            pltpu.make_async_copy(winv, win_h.at[pslot + b * 4 + r], sem).wait()
            plsc.subcore_barrier()
            for ra in range(4):
                pltpu.async_copy(minc_h.at[pslot + b * 4 + ra], minc_all.at[ra], sem)
                pltpu.async_copy(win_h.at[pslot + b * 4 + ra], win_all.at[ra], sem)
            for ra in range(4):
                pltpu.make_async_copy(minc_h.at[pslot + b * 4 + ra], minc_all.at[ra], sem).wait()
                pltpu.make_async_copy(win_h.at[pslot + b * 4 + ra], win_all.at[ra], sem).wait()
            cnt2 = jnp.int32(0)
            for ci in range(NCH):
                sl = pl.ds(ci * 16, 16)
                mm = minc_all[0, sl]
                ww = win_all[0, sl]
                for ra in range(1, 4):
                    mo = minc_all[ra, sl]
                    wo = win_all[ra, sl]
                    gt = mm > mo
                    eq = mm == mo
                    ww = jnp.where(gt, ww,
                                   jnp.where(eq, jnp.minimum(ww, wo), wo))
                    mm = jnp.maximum(mm, mo)
                h = ww < N
                old = ainv[sl]
                ev = h & (old >= 0)
                plsc.store_scatter(assign, [jnp.maximum(old, 0)], neg1,
                                   mask=ev)
                idxc = lane + (16 * ci)
                plsc.store_scatter(assign, [jnp.where(h, ww, 0)], idxc,
                                   mask=h)
                ainv[sl] = jnp.where(h, ww, old)
                price[sl] = price[sl] + jnp.where(h, mm, jnp.float32(0.0))
            for ci in range(NCH):
                sl = pl.ds(ci * 16, 16)
                u = assign[sl] < 0
                lc = lax.cumsum(u.astype(jnp.int32))
                plsc.store_scatter(lista, [cnt2 + lc - 1],
                                   lane + (16 * ci), mask=u)
                cnt2 = cnt2 + jnp.max(lc)
            return cnt2

        lax.fori_loop(0, iters_s, iter_body, jnp.int32(N))

        @pl.when(r == 0)
        def _():
            for ci in range(NCH):
                sl = pl.ds(ci * 16, 16)
                a = assign[sl]
                idx = jnp.maximum(a, 0)
                gx = plsc.load_gather(x2l, [idx])
                gy = plsc.load_gather(x2l, [idx + N])
                gz = plsc.load_gather(x2l, [idx + 2 * N])
                dx = x1l[sl] - gx
                dy = x1l[pl.ds(N + ci * 16, 16)] - gy
                dz = x1l[pl.ds(2 * N + ci * 16, 16)] - gz
                dd = (dx * dx + dy * dy) + dz * dz
                distv[sl] = jnp.where(a < 0, jnp.float32(0.0), dd)
            pltpu.sync_copy(distv, dist_o.at[b])
            pltpu.sync_copy(assign, ass_o.at[b])

    return k(negd, x1f, x2f, epsv, itersv)


def kernel(input1, input2, ass, ass_inv, price, eps, iters):
    x1 = input1.astype(jnp.float32)
    x2 = input2.astype(jnp.float32)
    d2 = (jnp.sum(x1 * x1, axis=-1)[:, :, None]
          + jnp.sum(x2 * x2, axis=-1)[:, None, :]
          - 2.0 * jnp.einsum('bnd,bmd->bnm', x1, x2))
    D = jnp.maximum(d2, 0.0)
    negd = (-D).reshape(B * N, N)
    x1f = jnp.transpose(x1, (0, 2, 1)).reshape(B, 3 * N)
    x2f = jnp.transpose(x2, (0, 2, 1)).reshape(B, 3 * N)
    epsv = jnp.full((16,), eps, jnp.float32)
    itersv = jnp.full((16,), iters, jnp.int32)
    dist, assignment, _, _ = _sc_auction(negd, x1f, x2f, epsv, itersv)
    return dist, assignment
